# baseline (speedup 1.0000x reference)
import functools

import jax
import jax.numpy as jnp
import numpy as np
from jax import lax
from jax.experimental import pallas as pl
from jax.experimental.pallas import tpu as pltpu
from jax.experimental.pallas import tpu_sc as plsc

N = 10000
E = 320000
G = 200
H = 128
HA = 112
PW = 240
P = 256
NC = 2
NS = 16
EPT = E // NS
CH = 80
NCK = EPT // CH
NPA = 632
NH = 5000
NPH = 5120
NB = 10
BN = N // NB
BE = 2000
NEB = E // BE
NG = 256
F32 = jnp.float32


def _leaky(x):
    return jnp.maximum(x, 0.01 * x)


def _elu(x):
    return jnp.where(x >= 0, x, jnp.exp(x) - 1.0)


def _tca_body(nf_ref, wpnT, bpn, wnT, wab, hv_ref, npc_ref, uv_ref):
    nf = nf_ref[...]
    hv = _leaky(nf @ wpnT[...] + bpn[...])
    hv_ref[...] = jnp.pad(hv, ((0, 0), (0, 56)))
    npm = nf @ wnT[...]
    np240 = jnp.pad(npm, ((0, 0), (0, PW - G)))
    npc_ref[...] = jnp.stack([np240[:, :H], np240[:, PW - H:]], axis=0)
    q = lax.dot_general(wab[...], hv, (((0,), (1,)), ((), ())),
                        preferred_element_type=F32)
    uv_ref[...] = jnp.concatenate([q[0:1], q[1:2]], axis=1)[None]


def _tca(nf, wpnT, bpn, wnT, wab):
    return pl.pallas_call(
        _tca_body,
        grid=(NB,),
        in_specs=[
            pl.BlockSpec((BN, 128), lambda i: (i, 0)),
            pl.BlockSpec((128, G), lambda i: (0, 0)),
            pl.BlockSpec((1, G), lambda i: (0, 0)),
            pl.BlockSpec((128, G), lambda i: (0, 0)),
            pl.BlockSpec((G, 2), lambda i: (0, 0)),
        ],
        out_specs=[
            pl.BlockSpec((BN, 256), lambda i: (i, 0)),
            pl.BlockSpec((NC, BN, H), lambda i: (0, i, 0)),
            pl.BlockSpec((1, 1, 2 * BN), lambda i: (i, 0, 0)),
        ],
        out_shape=[
            jax.ShapeDtypeStruct((N, 256), F32),
            jax.ShapeDtypeStruct((NC, N, H), F32),
            jax.ShapeDtypeStruct((NB, 1, 2 * BN), F32),
        ],
        compiler_params=pltpu.CompilerParams(
            dimension_semantics=("arbitrary",)),
    )(nf, wpnT, bpn, wnT, wab)


def _tcc_body(ef_ref, npg_ref, wefT, b1, waugT, baug, et_ref, sb_ref):
    ef = ef_ref[...]
    ep = ef @ wefT[...] + b1[...]
    npg = npg_ref[...]
    np240 = jnp.concatenate([npg[0][:, :PW - H], npg[1]], axis=1)
    he1 = _leaky(np240[:, :G] + ep)
    he256 = jnp.pad(he1, ((0, 0), (0, P - G)))
    eta = he256 @ waugT[...] + baug[...]
    et_ref[...] = jnp.stack([eta[:, :H], eta[:, PW - H:]], axis=0)
    sb_ref[...] = eta[:, 222][None, None]


def _tcc(ef, npg, wefT, b1, waugT, baug):
    return pl.pallas_call(
        _tcc_body,
        grid=(NEB,),
        in_specs=[
            pl.BlockSpec((BE, 16), lambda i: (i, 0)),
            pl.BlockSpec((NC, BE, H), lambda i: (0, i, 0)),
            pl.BlockSpec((16, G), lambda i: (0, 0)),
            pl.BlockSpec((1, G), lambda i: (0, 0)),
            pl.BlockSpec((P, PW), lambda i: (0, 0)),
            pl.BlockSpec((1, PW), lambda i: (0, 0)),
        ],
        out_specs=[
            pl.BlockSpec((NC, BE, H), lambda i: (0, i, 0)),
            pl.BlockSpec((1, 1, BE), lambda i: (i, 0, 0)),
        ],
        out_shape=[
            jax.ShapeDtypeStruct((NC, E, H), F32),
            jax.ShapeDtypeStruct((NEB, 1, BE), F32),
        ],
        compiler_params=pltpu.CompilerParams(
            dimension_semantics=("arbitrary",)),
    )(ef, npg, wefT, b1, waugT, baug)


def _tcd_body(parts_ref, h_ref, wihT, whhT, bih, bhh, wpnT, bpn, wab, b2,
              ho_ref, hpc_ref, uv_ref):
    pr = parts_ref[...][:, 0]
    nums = jnp.concatenate([pr[0][:, :PW - H], pr[1]], axis=1)
    ssum = nums[:, 223:224]
    ctx = _elu(nums[:, :G] / jnp.maximum(ssum, 1e-12))
    h = h_ref[...][:, :G]
    gi = ctx @ wihT[...] + bih[...]
    gh = h @ whhT[...] + bhh[...]
    r = jax.nn.sigmoid(gi[:, :G] + gh[:, :G])
    z = jax.nn.sigmoid(gi[:, G:2 * G] + gh[:, G:2 * G])
    nn_ = jnp.tanh(gi[:, 2 * G:] + r * gh[:, 2 * G:])
    hn = jax.nn.relu((1.0 - z) * nn_ + z * h)
    ho_ref[...] = jnp.pad(hn, ((0, 0), (0, 56)))
    hp = hn @ wpnT[...] + bpn[...]
    hpa = jnp.concatenate(
        [hp, jnp.zeros((hn.shape[0], 2 * HA - G - 1), F32),
         jnp.ones((hn.shape[0], 1), F32),
         jnp.zeros((hn.shape[0], PW - 2 * HA), F32)],
        axis=1)
    hpc_ref[...] = jnp.stack([hpa[:, :H], hpa[:, PW - H:]], axis=0)
    q = (lax.dot_general(wab[...], hn, (((0,), (1,)), ((), ())),
                         preferred_element_type=F32) + b2[...])
    uv_ref[...] = jnp.concatenate([q[0:1], q[1:2]], axis=1)[None]


def _tcd(parts, h, wihT, whhT, bih, bhh, wpnT, bpn, wab, b2):
    return pl.pallas_call(
        _tcd_body,
        grid=(NB,),
        in_specs=[
            pl.BlockSpec((NC, 1, BN, H), lambda i: (0, i // 5, i % 5, 0)),
            pl.BlockSpec((BN, 256), lambda i: (i, 0)),
            pl.BlockSpec((G, 3 * G), lambda i: (0, 0)),
            pl.BlockSpec((G, 3 * G), lambda i: (0, 0)),
            pl.BlockSpec((1, 3 * G), lambda i: (0, 0)),
            pl.BlockSpec((1, 3 * G), lambda i: (0, 0)),
            pl.BlockSpec((G, G), lambda i: (0, 0)),
            pl.BlockSpec((1, G), lambda i: (0, 0)),
            pl.BlockSpec((G, 2), lambda i: (0, 0)),
            pl.BlockSpec((2, 1), lambda i: (0, 0)),
        ],
        out_specs=[
            pl.BlockSpec((BN, 256), lambda i: (i, 0)),
            pl.BlockSpec((NC, BN, H), lambda i: (0, i, 0)),
            pl.BlockSpec((1, 1, 2 * BN), lambda i: (i, 0, 0)),
        ],
        out_shape=[
            jax.ShapeDtypeStruct((N, 256), F32),
            jax.ShapeDtypeStruct((NC, N, H), F32),
            jax.ShapeDtypeStruct((NB, 1, 2 * BN), F32),
        ],
        compiler_params=pltpu.CompilerParams(
            dimension_semantics=("arbitrary",)),
    )(parts, h, wihT, whhT, bih, bhh, wpnT, bpn, wab, b2)


def _tcr_body(ids_ref, h_ref, wa2, wb2, bt2, wpnT2, bpn2, wihT2, whhT2,
              bih2, bhh2, w1T, b1, w2Tp, b2p, out_ref, g_sc, an_sc, as_sc):
    p = pl.program_id(0)
    b = pl.program_id(1)
    h = h_ref[...][:, :G]
    ids = ids_ref[0, 0]
    M = (ids[:, None] == lax.broadcasted_iota(jnp.int32, (BN, NG), 1)
         ).astype(F32)

    @pl.when(jnp.logical_and(p == 0, b == 0))
    def _():
        g_sc[...] = jnp.zeros((NG, 256), F32)

    @pl.when(p == 0)
    def _():
        gb = lax.dot_general(M, h, (((0,), (0,)), ((), ())),
                             preferred_element_type=F32)
        g_sc[...] += jnp.pad(gb, ((0, 0), (0, 56)))

    @pl.when(p > 0)
    def _():
        t_is0 = p == 1

        def pick(w2):
            w = w2[...]
            return jnp.where(t_is0, w[0], w[1])

        wa = pick(wa2)
        wb = pick(wb2)
        bt = pick(bt2)
        wpnT = pick(wpnT2)
        bpn = pick(bpn2)

        @pl.when(b == 0)
        def _():
            an_sc[...] = jnp.zeros((NG, 256), F32)
            as_sc[...] = jnp.zeros((NG, 8), F32)

        g = g_sc[...][:, :G]
        ra = jax.nn.relu(g) @ wa
        raN = M @ ra[:, 0:1]
        zlog = _leaky(raN + (h @ wb)[:, 0:1] + bt[0, 0])
        ez = jnp.exp(zlog)
        hvp = h @ wpnT + bpn
        an_sc[...] += jnp.pad(
            lax.dot_general(M, ez * hvp, (((0,), (0,)), ((), ())),
                            preferred_element_type=F32),
            ((0, 0), (0, 56)))
        as_sc[:, 0:1] += lax.dot_general(M, ez, (((0,), (0,)), ((), ())),
                                         preferred_element_type=F32)

        @pl.when(b == NB - 1)
        def _():
            wihT = pick(wihT2)
            whhT = pick(whhT2)
            bih = pick(bih2)
            bhh = pick(bhh2)
            s = jnp.maximum(as_sc[...][:, 0:1], 1e-12)
            grp = an_sc[...][:, :G] / s
            gin = _elu(grp)
            gprev = g_sc[...][:, :G]
            gi = gin @ wihT + bih
            gh = gprev @ whhT + bhh
            r = jax.nn.sigmoid(gi[:, :G] + gh[:, :G])
            zz = jax.nn.sigmoid(gi[:, G:2 * G] + gh[:, G:2 * G])
            nn_ = jnp.tanh(gi[:, 2 * G:] + r * gh[:, 2 * G:])
            gnew = jax.nn.relu((1.0 - zz) * nn_ + zz * gprev)
            g_sc[...] = jnp.pad(gnew, ((0, 0), (0, 56)))

            @pl.when(p == 2)
            def _():
                hidden = jax.nn.relu(gnew @ w1T[...] + b1[...])
                out_ref[...] = hidden @ w2Tp[...] + b2p[...]


def _tcr(ids3, h, args):
    def full(s):
        return pl.BlockSpec(s, lambda p, b: tuple(0 for _ in s))

    return pl.pallas_call(
        _tcr_body,
        grid=(3, NB),
        in_specs=[
            pl.BlockSpec((1, 1, BN), lambda p, b: (b, 0, 0)),
            pl.BlockSpec((BN, 256), lambda p, b: (b, 0)),
            full((2, G, 8)), full((2, G, 8)), full((2, 1, 8)),
            full((2, G, G)), full((2, 1, G)),
            full((2, G, 3 * G)), full((2, G, 3 * G)),
            full((2, 1, 3 * G)), full((2, 1, 3 * G)),
            full((G, 1024)), full((1, 1024)),
            full((1024, 128)), full((1, 128)),
        ],
        out_specs=pl.BlockSpec((NG, 128), lambda p, b: (0, 0)),
        out_shape=jax.ShapeDtypeStruct((NG, 128), F32),
        scratch_shapes=[
            pltpu.VMEM((NG, 256), F32),
            pltpu.VMEM((NG, 256), F32),
            pltpu.VMEM((NG, 8), F32),
        ],
        compiler_params=pltpu.CompilerParams(
            dimension_semantics=("arbitrary", "arbitrary")),
    )(ids3, h, *args)


_MESH = plsc.VectorSubcoreMesh(core_axis_name="c", subcore_axis_name="s")


@functools.partial(
    pl.kernel,
    out_type=jax.ShapeDtypeStruct((NC, E, H), F32),
    mesh=_MESH,
    compiler_params=pltpu.CompilerParams(needs_layout_passes=False),
    scratch_types=[
        pltpu.VMEM((NCK, CH), jnp.int32),
        pltpu.VMEM((CH, H), F32),
        pltpu.VMEM((CH, H), F32),
        pltpu.SemaphoreType.DMA,
        pltpu.SemaphoreType.DMA,
    ],
)
def _sc_gather(srcp, tab, out, idx_v, bufA, bufB, semA, semB):
    c = lax.axis_index("c")
    s = lax.axis_index("s")
    pltpu.sync_copy(srcp.at[s], idx_v)
    cN = (c * N).astype(jnp.int32)

    def adj(j, _):
        for k in range(CH // 16):
            sl = (j, pl.ds(k * 16, 16))
            idx_v[sl] = idx_v[sl] + cN
        return 0

    lax.fori_loop(0, NCK, adj, 0, unroll=False)
    ebase = s * EPT

    def issue(j, buf, sem):
        pltpu.async_copy(tab.at[idx_v.at[j]], buf, sem)

    def wait(buf, sem):
        pltpu.make_async_copy(tab.at[pl.ds(0, CH)], buf, sem).wait()

    def flush(j, buf):
        pltpu.sync_copy(buf, out.at[c, pl.ds(ebase + j * CH, CH)])

    issue(0, bufA, semA)
    issue(1, bufB, semB)

    def body(jj, _):
        j0 = 2 * jj
        wait(bufA, semA)
        flush(j0, bufA)

        @pl.when(j0 + 2 < NCK)
        def _():
            issue(j0 + 2, bufA, semA)

        wait(bufB, semB)
        flush(j0 + 1, bufB)

        @pl.when(j0 + 3 < NCK)
        def _():
            issue(j0 + 3, bufB, semB)

        return 0

    lax.fori_loop(0, NCK // 2, body, 0, unroll=False)


_SPMM_SCRATCH = [
    pltpu.VMEM((NCK, CH), jnp.int32),
    pltpu.VMEM((2, CH), F32),
    pltpu.VMEM((2, CH), jnp.int32),
    pltpu.VMEM((2, CH), jnp.int32),
    pltpu.VMEM((N + 16,), F32),
    pltpu.VMEM((N + 16,), F32),
    pltpu.VMEM((2 * BN + 16,), F32),
    pltpu.VMEM((16,), jnp.int32),
    pltpu.VMEM((CH,), F32),
    pltpu.VMEM((CH,), F32),
    pltpu.VMEM((CH, H), F32),
    pltpu.VMEM((CH, H), F32),
    pltpu.VMEM_SHARED((NPH, H), F32),
    pltpu.SemaphoreType.DMA,
    pltpu.SemaphoreType.DMA,
    pltpu.SemaphoreType.DMA,
    pltpu.SemaphoreType.DMA,
    pltpu.SemaphoreType.DMA,
]


@functools.partial(
    pl.kernel,
    out_type=jax.ShapeDtypeStruct((NC, 2, NPH, H), F32),
    mesh=_MESH,
    compiler_params=pltpu.CompilerParams(needs_layout_passes=False),
    scratch_types=_SPMM_SCRATCH,
)
def _spmm(ridx, wlin, flg, dstp, uvp, tab, out,
          w_v, wl2, dst2, dsth2, u_v, v_v, uvr_v, fl_v, eA, eB, bufA, bufB,
          acc, semA, semB, semI, semSA, semSB):
    c = lax.axis_index("c")
    s = lax.axis_index("s")
    lane = lax.iota(jnp.int32, 16)
    zf16 = (lane * 0).astype(F32)
    pltpu.sync_copy(ridx.at[s], w_v)
    pltpu.sync_copy(flg, fl_v)
    for b in range(NB):
        pltpu.sync_copy(uvp.at[b, 0], uvr_v.at[pl.ds(0, 2 * BN)])
        for k in range(BN // 16 + 1):
            u_v[pl.ds(b * BN + k * 16, 16)] = uvr_v[pl.ds(k * 16, 16)]
            v_v[pl.ds(b * BN + k * 16, 16)] = uvr_v[pl.ds(BN + k * 16, 16)]
    cE = (c * E).astype(jnp.int32)

    def adj(j, _):
        for k in range(CH // 16):
            sl = (j, pl.ds(k * 16, 16))
            w_v[sl] = w_v[sl] + cE
        return 0

    lax.fori_loop(0, NCK, adj, 0, unroll=False)

    def issue(j, pb, buf, sem):
        pltpu.async_copy(tab.at[w_v.at[j]], buf, sem)
        pltpu.async_copy(dstp.at[s, j], dst2.at[pb], semI)
        pltpu.async_copy(wlin.at[s, j], wl2.at[pb], semI)

    def wait(pb, buf, sem):
        pltpu.make_async_copy(tab.at[pl.ds(0, CH)], buf, sem).wait()
        pltpu.make_async_copy(dstp.at[s, 0], dst2.at[pb], semI).wait()
        pltpu.make_async_copy(wlin.at[s, 0], wl2.at[pb], semI).wait()

    row0 = s * (NPH // NS)
    for hf in range(2):
        base = hf * NH

        def zr(i, _):
            for m in range(H // 16):
                bufA[i, pl.ds(m * 16, 16)] = zf16
            return 0

        lax.fori_loop(0, CH, zr, 0, unroll=False)
        for r in range(4):
            pltpu.sync_copy(bufA, acc.at[pl.ds(row0 + r * CH, CH)])
        plsc.subcore_barrier()

        def process(j, pb, buf, eb, semS):
            fv = fl_v[...] > 0
            for k in range(CH // 16):
                slk = pl.ds(k * 16, 16)
                idd = dst2[pb, slk]
                uu = plsc.load_gather(u_v, [idd])
                gidx = jnp.minimum(w_v[j, slk] - cE, N - 1)
                vg = plsc.load_gather(v_v, [gidx])
                vv = jnp.where(fv, wl2[pb, slk], vg)
                lg = uu + vv
                eb[slk] = jnp.exp(jnp.maximum(lg, 0.01 * lg))
                d = idd - base
                ok = (d >= 0) & (d < NH)
                dsth2[pb, slk] = jnp.where(ok, d, NPH - 1)

            def srow(i, _):
                ei = plsc.load_gather(
                    eb, [jnp.broadcast_to(i, (16,)).astype(jnp.int32)])
                for m in range(H // 16):
                    sl = (i, pl.ds(m * 16, 16))
                    buf[sl] = buf[sl] * ei
                return 0

            lax.fori_loop(0, CH, srow, 0, unroll=False)
            pltpu.async_copy(buf, acc.at[dsth2.at[pb]], semS, add=True)

        issue(0, 0, bufA, semA)
        issue(1, 1, bufB, semB)

        def swait(buf, semS):
            pltpu.make_async_copy(buf, acc.at[pl.ds(0, CH)], semS).wait()

        def body(jj, _):
            j0 = 2 * jj
            wait(0, bufA, semA)
            process(j0, 0, bufA, eA, semSA)

            wait(1, bufB, semB)
            process(j0 + 1, 1, bufB, eB, semSB)

            @pl.when(j0 + 2 < NCK)
            def _():
                swait(bufA, semSA)
                issue(j0 + 2, 0, bufA, semA)

            @pl.when(j0 + 3 < NCK)
            def _():
                swait(bufB, semSB)
                issue(j0 + 3, 1, bufB, semB)

            return 0

        lax.fori_loop(0, NCK // 2, body, 0, unroll=False)
        swait(bufA, semSA)
        swait(bufB, semSB)
        plsc.subcore_barrier()
        for r in range(4):
            pltpu.sync_copy(acc.at[pl.ds(row0 + r * CH, CH)], bufA)
            pltpu.sync_copy(bufA, out.at[c, hf, pl.ds(row0 + r * CH, CH)])
        plsc.subcore_barrier()


def _row(x):
    return x.reshape(1, -1)


def kernel(node_feat, edge_feat, edge_index, node_graph_ids, params):
    p = params
    src = edge_index[0]
    dst = edge_index[1]
    srcp = src.reshape(NS, NCK, CH)
    dstp = dst.reshape(NS, NCK, CH)
    eidp = jnp.arange(E, dtype=jnp.int32).reshape(NS, NCK, CH)

    wa_ctx = p['ctx_W_pe2'][0, :G]
    wb_ctx = p['ctx_W_pe2'][0, G:]
    b_ctx2 = p['ctx_b_pe2'][0]
    wab_ctx = jnp.stack([wa_ctx, jnp.zeros((G,), F32)], axis=1)
    waugT = jnp.zeros((P, PW), F32)
    waugT = waugT.at[:G, :G].set(p['ctx_W_et'].T)
    waugT = waugT.at[:G, 222].set(wb_ctx)
    baug = jnp.zeros((PW,), F32)
    baug = baug.at[:G].set(p['ctx_b_et'])
    baug = baug.at[222].set(b_ctx2)
    baug = baug.at[223].set(1.0)

    hv, npc, uc = _tca(node_feat,
                       p['ctx_W_pn'].T, _row(p['ctx_b_pn']),
                       p['ctx_W_pe1'][:, :128].T, wab_ctx)
    npg = _sc_gather(srcp, npc.reshape(NC * N, H))
    et, sbp = _tcc(edge_feat, npg,
                   p['ctx_W_pe1'][:, 128:].T, _row(p['ctx_b_pe1']),
                   waugT, _row(baug))
    sbp = sbp.reshape(NS, NCK, CH)

    ridx_s = jnp.concatenate([eidp[None]] + [srcp[None]] * 4, axis=0)
    wlin_s = jnp.concatenate([sbp[None], jnp.zeros((4, NS, NCK, CH), F32)], axis=0)
    flg_s = jnp.concatenate([jnp.ones((1, 16), jnp.int32),
                             jnp.zeros((4, 16), jnp.int32)], axis=0)

    def stk(key_c, key_g, tr):
        c = p[key_c].T if tr else p[key_c]
        gs = [p[key_g][l].T if tr else p[key_g][l] for l in range(4)]
        return jnp.stack([c] + gs, axis=0)

    wihT_s = stk('ctx_Wih', 'gnn_Wih', True)
    whhT_s = stk('ctx_Whh', 'gnn_Whh', True)
    bih_s = jnp.stack([_row(p['ctx_bih'])] + [_row(p['gnn_bih'][l]) for l in range(4)])
    bhh_s = jnp.stack([_row(p['ctx_bhh'])] + [_row(p['gnn_bhh'][l]) for l in range(4)])

    def prep(l):
        wa = p['gnn_W_pe'][l][0, :G]
        wb = p['gnn_W_pe'][l][0, G:]
        b = p['gnn_b_pe'][l][0]
        wab = jnp.stack([wa, wb], axis=1)
        b2 = jnp.stack([jnp.zeros((), F32), b]).reshape(2, 1)
        return p['gnn_W_pn'][l].T, _row(p['gnn_b_pn'][l]), wab, b2

    preps = [prep(l) for l in [0, 1, 2, 3, 0]]
    wpnT_s = jnp.stack([q[0] for q in preps])
    bpn_s = jnp.stack([q[1] for q in preps])
    wab_s = jnp.stack([q[2] for q in preps])
    b2_s = jnp.stack([q[3] for q in preps])

    tab0 = et.reshape(NC * E, H)

    def step(carry, xs):
        h, uvp, tab = carry
        ridx, wlin, flg, wihT, whhT, bih, bhh, wpnT, bpn, wab, b2 = xs
        parts = _spmm(ridx, wlin, flg, dstp, uvp, tab)
        h2, hpc, uv = _tcd(parts, h, wihT, whhT, bih, bhh, wpnT, bpn, wab, b2)
        tab = lax.dynamic_update_slice(tab, hpc[0], (0, 0))
        tab = lax.dynamic_update_slice(tab, hpc[1], (E, 0))
        return (h2, uv, tab), 0.0

    (h, _, _), _ = lax.scan(
        step, (hv, uc, tab0),
        (ridx_s, wlin_s, flg_s, wihT_s, whhT_s, bih_s, bhh_s,
         wpnT_s, bpn_s, wab_s, b2_s))

    def pad8(v):
        return jnp.pad(v.reshape(-1, 1), ((0, 0), (0, 7)))

    ids3 = node_graph_ids.reshape(NB, 1, BN)
    ro_args = (
        jnp.stack([pad8(p['ro_W_cl'][t][0, :G]) for t in range(2)]),
        jnp.stack([pad8(p['ro_W_cl'][t][0, G:]) for t in range(2)]),
        jnp.stack([pad8(p['ro_b_cl'][t]) for t in range(2)]),
        jnp.stack([p['ro_W_pn'][t].T for t in range(2)]),
        jnp.stack([_row(p['ro_b_pn'][t]) for t in range(2)]),
        jnp.stack([p['ro_Wih'][t].T for t in range(2)]),
        jnp.stack([p['ro_Whh'][t].T for t in range(2)]),
        jnp.stack([_row(p['ro_bih'][t]) for t in range(2)]),
        jnp.stack([_row(p['ro_bhh'][t]) for t in range(2)]),
        p['out_W1'].T, _row(p['out_b1']),
        jnp.pad(p['out_W2'].T, ((0, 0), (0, 127))),
        jnp.pad(_row(p['out_b2']), ((0, 0), (0, 127))),
    )
    out128 = _tcr(ids3, h, ro_args)
    return out128[:, :1]


# speedup vs baseline: 1.6507x; 1.6507x over previous
import functools

import jax
import jax.numpy as jnp
import numpy as np
from jax import lax
from jax.experimental import pallas as pl
from jax.experimental.pallas import tpu as pltpu
from jax.experimental.pallas import tpu_sc as plsc

N = 10000
E = 320000
G = 200
H = 128
HA = 112
PW = 240
P = 256
NC = 2
NS = 16
EPT = E // NS
CH = 80
NCK = EPT // CH
NPA = 632
NH = 5000
NPH = 5120
NB = 10
BN = N // NB
BE = 2000
NEB = E // BE
NG = 256
F32 = jnp.float32


def _leaky(x):
    return jnp.maximum(x, 0.01 * x)


def _elu(x):
    return jnp.where(x >= 0, x, jnp.exp(x) - 1.0)


def _tca_body(nf_ref, wpnT, bpn, wnT, wab, hv_ref, npc_ref, uv_ref):
    nf = nf_ref[...]
    hv = _leaky(nf @ wpnT[...] + bpn[...])
    hv_ref[...] = jnp.pad(hv, ((0, 0), (0, 56)))
    npm = nf @ wnT[...]
    np240 = jnp.pad(npm, ((0, 0), (0, PW - G)))
    npc_ref[...] = jnp.stack([np240[:, :H], np240[:, PW - H:]], axis=0)
    q = lax.dot_general(wab[...], hv, (((0,), (1,)), ((), ())),
                        preferred_element_type=F32)
    uv_ref[...] = jnp.concatenate([q[0:1], q[1:2]], axis=1)[None]


def _tca(nf, wpnT, bpn, wnT, wab):
    return pl.pallas_call(
        _tca_body,
        grid=(NB,),
        in_specs=[
            pl.BlockSpec((BN, 128), lambda i: (i, 0)),
            pl.BlockSpec((128, G), lambda i: (0, 0)),
            pl.BlockSpec((1, G), lambda i: (0, 0)),
            pl.BlockSpec((128, G), lambda i: (0, 0)),
            pl.BlockSpec((G, 2), lambda i: (0, 0)),
        ],
        out_specs=[
            pl.BlockSpec((BN, 256), lambda i: (i, 0)),
            pl.BlockSpec((NC, BN, H), lambda i: (0, i, 0)),
            pl.BlockSpec((1, 1, 2 * BN), lambda i: (i, 0, 0)),
        ],
        out_shape=[
            jax.ShapeDtypeStruct((N, 256), F32),
            jax.ShapeDtypeStruct((NC, N, H), F32),
            jax.ShapeDtypeStruct((NB, 1, 2 * BN), F32),
        ],
        compiler_params=pltpu.CompilerParams(
            dimension_semantics=("arbitrary",)),
    )(nf, wpnT, bpn, wnT, wab)


def _tcc_body(ef_ref, npg_ref, wefT, b1, waugT, baug, et_ref, sb_ref):
    ef = ef_ref[...]
    ep = ef @ wefT[...] + b1[...]
    npg = npg_ref[...]
    np240 = jnp.concatenate([npg[0][:, :PW - H], npg[1]], axis=1)
    he1 = _leaky(np240[:, :G] + ep)
    he256 = jnp.pad(he1, ((0, 0), (0, P - G)))
    eta = he256 @ waugT[...] + baug[...]
    et_ref[...] = jnp.stack([eta[:, :H], eta[:, PW - H:]], axis=0)
    sb_ref[...] = eta[:, 222][None, None]


def _tcc(ef, npg, wefT, b1, waugT, baug):
    return pl.pallas_call(
        _tcc_body,
        grid=(NEB,),
        in_specs=[
            pl.BlockSpec((BE, 16), lambda i: (i, 0)),
            pl.BlockSpec((NC, BE, H), lambda i: (0, i, 0)),
            pl.BlockSpec((16, G), lambda i: (0, 0)),
            pl.BlockSpec((1, G), lambda i: (0, 0)),
            pl.BlockSpec((P, PW), lambda i: (0, 0)),
            pl.BlockSpec((1, PW), lambda i: (0, 0)),
        ],
        out_specs=[
            pl.BlockSpec((NC, BE, H), lambda i: (0, i, 0)),
            pl.BlockSpec((1, 1, BE), lambda i: (i, 0, 0)),
        ],
        out_shape=[
            jax.ShapeDtypeStruct((NC, E, H), F32),
            jax.ShapeDtypeStruct((NEB, 1, BE), F32),
        ],
        compiler_params=pltpu.CompilerParams(
            dimension_semantics=("arbitrary",)),
    )(ef, npg, wefT, b1, waugT, baug)


def _tcd_body(parts_ref, h_ref, wihT, whhT, bih, bhh, wpnT, bpn, wab, b2,
              ho_ref, hpc_ref, uv_ref):
    pr = parts_ref[...]
    nums = jnp.concatenate([pr[0][:, :PW - H], pr[1]], axis=1)
    ssum = nums[:, 223:224]
    ctx = _elu(nums[:, :G] / jnp.maximum(ssum, 1e-12))
    h = h_ref[...][:, :G]
    gi = ctx @ wihT[...] + bih[...]
    gh = h @ whhT[...] + bhh[...]
    r = jax.nn.sigmoid(gi[:, :G] + gh[:, :G])
    z = jax.nn.sigmoid(gi[:, G:2 * G] + gh[:, G:2 * G])
    nn_ = jnp.tanh(gi[:, 2 * G:] + r * gh[:, 2 * G:])
    hn = jax.nn.relu((1.0 - z) * nn_ + z * h)
    ho_ref[...] = jnp.pad(hn, ((0, 0), (0, 56)))
    hp = hn @ wpnT[...] + bpn[...]
    hpa = jnp.concatenate(
        [hp, jnp.zeros((hn.shape[0], 2 * HA - G - 1), F32),
         jnp.ones((hn.shape[0], 1), F32),
         jnp.zeros((hn.shape[0], PW - 2 * HA), F32)],
        axis=1)
    hpc_ref[...] = jnp.stack([hpa[:, :H], hpa[:, PW - H:]], axis=0)
    q = (lax.dot_general(wab[...], hn, (((0,), (1,)), ((), ())),
                         preferred_element_type=F32) + b2[...])
    uv_ref[...] = jnp.concatenate([q[0:1], q[1:2]], axis=1)[None]


def _tcd(parts, h, wihT, whhT, bih, bhh, wpnT, bpn, wab, b2):
    return pl.pallas_call(
        _tcd_body,
        grid=(NB,),
        in_specs=[
            pl.BlockSpec((NC, BN, H), lambda i: (0, i, 0)),
            pl.BlockSpec((BN, 256), lambda i: (i, 0)),
            pl.BlockSpec((G, 3 * G), lambda i: (0, 0)),
            pl.BlockSpec((G, 3 * G), lambda i: (0, 0)),
            pl.BlockSpec((1, 3 * G), lambda i: (0, 0)),
            pl.BlockSpec((1, 3 * G), lambda i: (0, 0)),
            pl.BlockSpec((G, G), lambda i: (0, 0)),
            pl.BlockSpec((1, G), lambda i: (0, 0)),
            pl.BlockSpec((G, 2), lambda i: (0, 0)),
            pl.BlockSpec((2, 1), lambda i: (0, 0)),
        ],
        out_specs=[
            pl.BlockSpec((BN, 256), lambda i: (i, 0)),
            pl.BlockSpec((NC, BN, H), lambda i: (0, i, 0)),
            pl.BlockSpec((1, 1, 2 * BN), lambda i: (i, 0, 0)),
        ],
        out_shape=[
            jax.ShapeDtypeStruct((N, 256), F32),
            jax.ShapeDtypeStruct((NC, N, H), F32),
            jax.ShapeDtypeStruct((NB, 1, 2 * BN), F32),
        ],
        compiler_params=pltpu.CompilerParams(
            dimension_semantics=("arbitrary",)),
    )(parts, h, wihT, whhT, bih, bhh, wpnT, bpn, wab, b2)


def _tcr_body(ids_ref, h_ref, wa2, wb2, bt2, wpnT2, bpn2, wihT2, whhT2,
              bih2, bhh2, w1T, b1, w2Tp, b2p, out_ref, g_sc, an_sc, as_sc):
    p = pl.program_id(0)
    b = pl.program_id(1)
    h = h_ref[...][:, :G]
    ids = ids_ref[0, 0]
    M = (ids[:, None] == lax.broadcasted_iota(jnp.int32, (BN, NG), 1)
         ).astype(F32)

    @pl.when(jnp.logical_and(p == 0, b == 0))
    def _():
        g_sc[...] = jnp.zeros((NG, 256), F32)

    @pl.when(p == 0)
    def _():
        gb = lax.dot_general(M, h, (((0,), (0,)), ((), ())),
                             preferred_element_type=F32)
        g_sc[...] += jnp.pad(gb, ((0, 0), (0, 56)))

    @pl.when(p > 0)
    def _():
        t_is0 = p == 1

        def pick(w2):
            w = w2[...]
            return jnp.where(t_is0, w[0], w[1])

        wa = pick(wa2)
        wb = pick(wb2)
        bt = pick(bt2)
        wpnT = pick(wpnT2)
        bpn = pick(bpn2)

        @pl.when(b == 0)
        def _():
            an_sc[...] = jnp.zeros((NG, 256), F32)
            as_sc[...] = jnp.zeros((NG, 8), F32)

        g = g_sc[...][:, :G]
        ra = jax.nn.relu(g) @ wa
        raN = M @ ra[:, 0:1]
        zlog = _leaky(raN + (h @ wb)[:, 0:1] + bt[0, 0])
        ez = jnp.exp(zlog)
        hvp = h @ wpnT + bpn
        an_sc[...] += jnp.pad(
            lax.dot_general(M, ez * hvp, (((0,), (0,)), ((), ())),
                            preferred_element_type=F32),
            ((0, 0), (0, 56)))
        as_sc[:, 0:1] += lax.dot_general(M, ez, (((0,), (0,)), ((), ())),
                                         preferred_element_type=F32)

        @pl.when(b == NB - 1)
        def _():
            wihT = pick(wihT2)
            whhT = pick(whhT2)
            bih = pick(bih2)
            bhh = pick(bhh2)
            s = jnp.maximum(as_sc[...][:, 0:1], 1e-12)
            grp = an_sc[...][:, :G] / s
            gin = _elu(grp)
            gprev = g_sc[...][:, :G]
            gi = gin @ wihT + bih
            gh = gprev @ whhT + bhh
            r = jax.nn.sigmoid(gi[:, :G] + gh[:, :G])
            zz = jax.nn.sigmoid(gi[:, G:2 * G] + gh[:, G:2 * G])
            nn_ = jnp.tanh(gi[:, 2 * G:] + r * gh[:, 2 * G:])
            gnew = jax.nn.relu((1.0 - zz) * nn_ + zz * gprev)
            g_sc[...] = jnp.pad(gnew, ((0, 0), (0, 56)))

            @pl.when(p == 2)
            def _():
                hidden = jax.nn.relu(gnew @ w1T[...] + b1[...])
                out_ref[...] = hidden @ w2Tp[...] + b2p[...]


def _tcr(ids3, h, args):
    def full(s):
        return pl.BlockSpec(s, lambda p, b: tuple(0 for _ in s))

    return pl.pallas_call(
        _tcr_body,
        grid=(3, NB),
        in_specs=[
            pl.BlockSpec((1, 1, BN), lambda p, b: (b, 0, 0)),
            pl.BlockSpec((BN, 256), lambda p, b: (b, 0)),
            full((2, G, 8)), full((2, G, 8)), full((2, 1, 8)),
            full((2, G, G)), full((2, 1, G)),
            full((2, G, 3 * G)), full((2, G, 3 * G)),
            full((2, 1, 3 * G)), full((2, 1, 3 * G)),
            full((G, 1024)), full((1, 1024)),
            full((1024, 128)), full((1, 128)),
        ],
        out_specs=pl.BlockSpec((NG, 128), lambda p, b: (0, 0)),
        out_shape=jax.ShapeDtypeStruct((NG, 128), F32),
        scratch_shapes=[
            pltpu.VMEM((NG, 256), F32),
            pltpu.VMEM((NG, 256), F32),
            pltpu.VMEM((NG, 8), F32),
        ],
        compiler_params=pltpu.CompilerParams(
            dimension_semantics=("arbitrary", "arbitrary")),
    )(ids3, h, *args)


_MESH = plsc.VectorSubcoreMesh(core_axis_name="c", subcore_axis_name="s")


@functools.partial(
    pl.kernel,
    out_type=jax.ShapeDtypeStruct((NC, E, H), F32),
    mesh=_MESH,
    compiler_params=pltpu.CompilerParams(needs_layout_passes=False),
    scratch_types=[
        pltpu.VMEM((NCK, CH), jnp.int32),
        pltpu.VMEM((CH, H), F32),
        pltpu.VMEM((CH, H), F32),
        pltpu.SemaphoreType.DMA,
        pltpu.SemaphoreType.DMA,
    ],
)
def _sc_gather(srcp, tab, out, idx_v, bufA, bufB, semA, semB):
    c = lax.axis_index("c")
    s = lax.axis_index("s")
    pltpu.sync_copy(srcp.at[s], idx_v)
    cN = (c * N).astype(jnp.int32)

    def adj(j, _):
        for k in range(CH // 16):
            sl = (j, pl.ds(k * 16, 16))
            idx_v[sl] = idx_v[sl] + cN
        return 0

    lax.fori_loop(0, NCK, adj, 0, unroll=False)
    ebase = s * EPT

    def issue(j, buf, sem):
        pltpu.async_copy(tab.at[idx_v.at[j]], buf, sem)

    def wait(buf, sem):
        pltpu.make_async_copy(tab.at[pl.ds(0, CH)], buf, sem).wait()

    def flush(j, buf):
        pltpu.sync_copy(buf, out.at[c, pl.ds(ebase + j * CH, CH)])

    issue(0, bufA, semA)
    issue(1, bufB, semB)

    def body(jj, _):
        j0 = 2 * jj
        wait(bufA, semA)
        flush(j0, bufA)

        @pl.when(j0 + 2 < NCK)
        def _():
            issue(j0 + 2, bufA, semA)

        wait(bufB, semB)
        flush(j0 + 1, bufB)

        @pl.when(j0 + 3 < NCK)
        def _():
            issue(j0 + 3, bufB, semB)

        return 0

    lax.fori_loop(0, NCK // 2, body, 0, unroll=False)


_SPMM_SCRATCH = [
    pltpu.VMEM((4, CH), jnp.int32),
    pltpu.VMEM((2, CH), F32),
    pltpu.VMEM((2, CH), jnp.int32),
    pltpu.VMEM((N + BN + 16,), F32),
    pltpu.VMEM((N + BN + 16,), F32),
    pltpu.VMEM((16,), jnp.int32),
    pltpu.VMEM((CH,), F32),
    pltpu.VMEM((CH, H), F32),
    pltpu.VMEM((CH, H), F32),
    pltpu.VMEM_SHARED((N, H), F32),
    pltpu.SemaphoreType.DMA,
    pltpu.SemaphoreType.DMA,
    pltpu.SemaphoreType.DMA,
    pltpu.SemaphoreType.DMA,
]


@functools.partial(
    pl.kernel,
    out_type=jax.ShapeDtypeStruct((NC, N, H), F32),
    mesh=_MESH,
    compiler_params=pltpu.CompilerParams(needs_layout_passes=False),
    scratch_types=_SPMM_SCRATCH,
)
def _spmm(ridx, wlin, flg, dstp, uvp, tab, out,
          idxb, wl2, dst2, u_v, v_v, fl_v, e_v, bufA, bufB,
          acc, semA, semB, semI, semX):
    c = lax.axis_index("c")
    s = lax.axis_index("s")
    lane = lax.iota(jnp.int32, 16)
    zf16 = (lane * 0).astype(F32)
    pltpu.sync_copy(flg, fl_v)
    for b in range(NB):
        pltpu.sync_copy(uvp.at[b, 0], u_v.at[pl.ds(b * BN, 2 * BN)])
    for b in reversed(range(NB)):
        pltpu.sync_copy(uvp.at[b, 0], v_v.at[pl.ds(b * BN, 2 * BN)])
    cE = (c * E).astype(jnp.int32)

    def zr(i, _):
        for m in range(H // 16):
            bufA[i, pl.ds(m * 16, 16)] = zf16
        return 0

    lax.fori_loop(0, CH, zr, 0, unroll=False)

    @pl.when(s < NB)
    def _():
        for r in range(12):
            pltpu.sync_copy(bufA, acc.at[pl.ds(s * BN + r * CH, CH)])
        pltpu.sync_copy(bufA.at[pl.ds(0, 40)],
                        acc.at[pl.ds(s * BN + 12 * CH, 40)])
    plsc.subcore_barrier()

    def issue_idx(j):
        pltpu.async_copy(ridx.at[s, j], idxb.at[j % 4], semX)

    def issue_row(j, buf, sem):
        pltpu.make_async_copy(ridx.at[s, 0], idxb.at[j % 4], semX).wait()
        for k in range(CH // 16):
            sl = (j % 4, pl.ds(k * 16, 16))
            idxb[sl] = idxb[sl] + cE
        pltpu.async_copy(tab.at[idxb.at[j % 4]], buf, sem)
        pltpu.async_copy(dstp.at[s, j], dst2.at[j % 2], semI)
        pltpu.async_copy(wlin.at[s, j], wl2.at[j % 2], semI)

    def wait_row(j, buf, sem):
        pltpu.make_async_copy(tab.at[pl.ds(0, CH)], buf, sem).wait()
        pltpu.make_async_copy(dstp.at[s, 0], dst2.at[j % 2], semI).wait()
        pltpu.make_async_copy(wlin.at[s, 0], wl2.at[j % 2], semI).wait()

    def process(j, buf):
        pb = j % 2
        fv = fl_v[...] > 0
        for k in range(CH // 16):
            slk = pl.ds(k * 16, 16)
            idd = dst2[pb, slk]
            uu = plsc.load_gather(u_v, [idd])
            gidx = jnp.minimum(idxb[j % 4, slk] - cE, N - 1) + BN
            vg = plsc.load_gather(v_v, [gidx])
            vv = jnp.where(fv, wl2[pb, slk], vg)
            lg = uu + vv
            e_v[slk] = jnp.exp(jnp.maximum(lg, 0.01 * lg))

        def srow(i, _):
            ei = plsc.load_gather(
                e_v, [jnp.broadcast_to(i, (16,)).astype(jnp.int32)])
            for m in range(H // 16):
                sl = (i, pl.ds(m * 16, 16))
                buf[sl] = buf[sl] * ei
            return 0

        lax.fori_loop(0, CH, srow, 0, unroll=False)
        pltpu.sync_copy(buf, acc.at[dst2.at[pb]], add=True)

    issue_idx(0)
    issue_idx(1)
    issue_row(0, bufA, semA)
    issue_idx(2)
    issue_row(1, bufB, semB)
    issue_idx(3)

    def body(jj, _):
        j0 = 2 * jj
        wait_row(j0, bufA, semA)
        process(j0, bufA)

        @pl.when(j0 + 2 < NCK)
        def _():
            issue_row(j0 + 2, bufA, semA)

        @pl.when(j0 + 4 < NCK)
        def _():
            issue_idx(j0 + 4)

        wait_row(j0 + 1, bufB, semB)
        process(j0 + 1, bufB)

        @pl.when(j0 + 3 < NCK)
        def _():
            issue_row(j0 + 3, bufB, semB)

        @pl.when(j0 + 5 < NCK)
        def _():
            issue_idx(j0 + 5)

        return 0

    lax.fori_loop(0, NCK // 2, body, 0, unroll=False)
    plsc.subcore_barrier()

    @pl.when(s < NB)
    def _():
        for r in range(12):
            pltpu.sync_copy(acc.at[pl.ds(s * BN + r * CH, CH)], bufA)
            pltpu.sync_copy(bufA, out.at[c, pl.ds(s * BN + r * CH, CH)])
        pltpu.sync_copy(acc.at[pl.ds(s * BN + 12 * CH, 40)],
                        bufA.at[pl.ds(0, 40)])
        pltpu.sync_copy(bufA.at[pl.ds(0, 40)],
                        out.at[c, pl.ds(s * BN + 12 * CH, 40)])


def _row(x):
    return x.reshape(1, -1)


def kernel(node_feat, edge_feat, edge_index, node_graph_ids, params):
    p = params
    src = edge_index[0]
    dst = edge_index[1]
    srcp = src.reshape(NS, NCK, CH)
    dstp = dst.reshape(NS, NCK, CH)
    eidp = jnp.arange(E, dtype=jnp.int32).reshape(NS, NCK, CH)

    wa_ctx = p['ctx_W_pe2'][0, :G]
    wb_ctx = p['ctx_W_pe2'][0, G:]
    b_ctx2 = p['ctx_b_pe2'][0]
    wab_ctx = jnp.stack([wa_ctx, jnp.zeros((G,), F32)], axis=1)
    waugT = jnp.zeros((P, PW), F32)
    waugT = waugT.at[:G, :G].set(p['ctx_W_et'].T)
    waugT = waugT.at[:G, 222].set(wb_ctx)
    baug = jnp.zeros((PW,), F32)
    baug = baug.at[:G].set(p['ctx_b_et'])
    baug = baug.at[222].set(b_ctx2)
    baug = baug.at[223].set(1.0)

    hv, npc, uc = _tca(node_feat,
                       p['ctx_W_pn'].T, _row(p['ctx_b_pn']),
                       p['ctx_W_pe1'][:, :128].T, wab_ctx)
    npg = _sc_gather(srcp, npc.reshape(NC * N, H))
    et, sbp = _tcc(edge_feat, npg,
                   p['ctx_W_pe1'][:, 128:].T, _row(p['ctx_b_pe1']),
                   waugT, _row(baug))
    sbp = sbp.reshape(NS, NCK, CH)

    ridx_s = jnp.concatenate([eidp[None]] + [srcp[None]] * 4, axis=0)
    wlin_s = jnp.concatenate([sbp[None], jnp.zeros((4, NS, NCK, CH), F32)], axis=0)
    flg_s = jnp.concatenate([jnp.ones((1, 16), jnp.int32),
                             jnp.zeros((4, 16), jnp.int32)], axis=0)

    def stk(key_c, key_g, tr):
        c = p[key_c].T if tr else p[key_c]
        gs = [p[key_g][l].T if tr else p[key_g][l] for l in range(4)]
        return jnp.stack([c] + gs, axis=0)

    wihT_s = stk('ctx_Wih', 'gnn_Wih', True)
    whhT_s = stk('ctx_Whh', 'gnn_Whh', True)
    bih_s = jnp.stack([_row(p['ctx_bih'])] + [_row(p['gnn_bih'][l]) for l in range(4)])
    bhh_s = jnp.stack([_row(p['ctx_bhh'])] + [_row(p['gnn_bhh'][l]) for l in range(4)])

    def prep(l):
        wa = p['gnn_W_pe'][l][0, :G]
        wb = p['gnn_W_pe'][l][0, G:]
        b = p['gnn_b_pe'][l][0]
        wab = jnp.stack([wa, wb], axis=1)
        b2 = jnp.stack([jnp.zeros((), F32), b]).reshape(2, 1)
        return p['gnn_W_pn'][l].T, _row(p['gnn_b_pn'][l]), wab, b2

    preps = [prep(l) for l in [0, 1, 2, 3, 0]]
    wpnT_s = jnp.stack([q[0] for q in preps])
    bpn_s = jnp.stack([q[1] for q in preps])
    wab_s = jnp.stack([q[2] for q in preps])
    b2_s = jnp.stack([q[3] for q in preps])

    tab0 = et.reshape(NC * E, H)

    def step(carry, xs):
        h, uvp, tab = carry
        ridx, wlin, flg, wihT, whhT, bih, bhh, wpnT, bpn, wab, b2 = xs
        parts = _spmm(ridx, wlin, flg, dstp, uvp, tab)
        h2, hpc, uv = _tcd(parts, h, wihT, whhT, bih, bhh, wpnT, bpn, wab, b2)
        tab = lax.dynamic_update_slice(tab, hpc[0], (0, 0))
        tab = lax.dynamic_update_slice(tab, hpc[1], (E, 0))
        return (h2, uv, tab), 0.0

    (h, _, _), _ = lax.scan(
        step, (hv, uc, tab0),
        (ridx_s, wlin_s, flg_s, wihT_s, whhT_s, bih_s, bhh_s,
         wpnT_s, bpn_s, wab_s, b2_s))

    def pad8(v):
        return jnp.pad(v.reshape(-1, 1), ((0, 0), (0, 7)))

    ids3 = node_graph_ids.reshape(NB, 1, BN)
    ro_args = (
        jnp.stack([pad8(p['ro_W_cl'][t][0, :G]) for t in range(2)]),
        jnp.stack([pad8(p['ro_W_cl'][t][0, G:]) for t in range(2)]),
        jnp.stack([pad8(p['ro_b_cl'][t]) for t in range(2)]),
        jnp.stack([p['ro_W_pn'][t].T for t in range(2)]),
        jnp.stack([_row(p['ro_b_pn'][t]) for t in range(2)]),
        jnp.stack([p['ro_Wih'][t].T for t in range(2)]),
        jnp.stack([p['ro_Whh'][t].T for t in range(2)]),
        jnp.stack([_row(p['ro_bih'][t]) for t in range(2)]),
        jnp.stack([_row(p['ro_bhh'][t]) for t in range(2)]),
        p['out_W1'].T, _row(p['out_b1']),
        jnp.pad(p['out_W2'].T, ((0, 0), (0, 127))),
        jnp.pad(_row(p['out_b2']), ((0, 0), (0, 127))),
    )
    out128 = _tcr(ids3, h, ro_args)
    return out128[:, :1]


# speedup vs baseline: 1.6515x; 1.0005x over previous
import functools

import jax
import jax.numpy as jnp
import numpy as np
from jax import lax
from jax.experimental import pallas as pl
from jax.experimental.pallas import tpu as pltpu
from jax.experimental.pallas import tpu_sc as plsc

N = 10000
E = 320000
G = 200
H = 128
HA = 112
PW = 240
P = 256
NC = 2
NS = 16
EPT = E // NS
CH = 80
NCK = EPT // CH
NB = 10
BN = N // NB
BE = 2000
NEB = E // BE
NG = 256
F32 = jnp.float32


def _leaky(x):
    return jnp.maximum(x, 0.01 * x)


def _elu(x):
    return jnp.where(x >= 0, x, jnp.exp(x) - 1.0)


def _tca_body(nf_ref, wpnT, bpn, wnT, wab, hv_ref, npc_ref, uv_ref):
    nf = nf_ref[...]
    hv = _leaky(nf @ wpnT[...] + bpn[...])
    hv_ref[...] = jnp.pad(hv, ((0, 0), (0, 56)))
    npm = nf @ wnT[...]
    np240 = jnp.pad(npm, ((0, 0), (0, PW - G)))
    npc_ref[...] = jnp.stack([np240[:, :H], np240[:, PW - H:]], axis=0)
    q = lax.dot_general(wab[...], hv, (((0,), (1,)), ((), ())),
                        preferred_element_type=F32)
    uv_ref[...] = jnp.concatenate([q[0:1], q[1:2]], axis=1)[None]


def _tca(nf, wpnT, bpn, wnT, wab):
    return pl.pallas_call(
        _tca_body,
        grid=(NB,),
        in_specs=[
            pl.BlockSpec((BN, 128), lambda i: (i, 0)),
            pl.BlockSpec((128, G), lambda i: (0, 0)),
            pl.BlockSpec((1, G), lambda i: (0, 0)),
            pl.BlockSpec((128, G), lambda i: (0, 0)),
            pl.BlockSpec((G, 2), lambda i: (0, 0)),
        ],
        out_specs=[
            pl.BlockSpec((BN, 256), lambda i: (i, 0)),
            pl.BlockSpec((NC, BN, H), lambda i: (0, i, 0)),
            pl.BlockSpec((1, 1, 2 * BN), lambda i: (i, 0, 0)),
        ],
        out_shape=[
            jax.ShapeDtypeStruct((N, 256), F32),
            jax.ShapeDtypeStruct((NC, N, H), F32),
            jax.ShapeDtypeStruct((NB, 1, 2 * BN), F32),
        ],
        compiler_params=pltpu.CompilerParams(
            dimension_semantics=("arbitrary",)),
    )(nf, wpnT, bpn, wnT, wab)


def _tcc_body(ef_ref, npg_ref, wefT, b1, waugT, baug, et_ref, sb_ref):
    ef = ef_ref[...]
    ep = ef @ wefT[...] + b1[...]
    npg = npg_ref[...]
    np240 = jnp.concatenate([npg[0][:, :PW - H], npg[1]], axis=1)
    he1 = _leaky(np240[:, :G] + ep)
    he256 = jnp.pad(he1, ((0, 0), (0, P - G)))
    eta = he256 @ waugT[...] + baug[...]
    et_ref[...] = jnp.stack([eta[:, :H], eta[:, PW - H:]], axis=0)
    sb_ref[...] = eta[:, 222][None, None]


def _tcc(ef, npg, wefT, b1, waugT, baug):
    return pl.pallas_call(
        _tcc_body,
        grid=(NEB,),
        in_specs=[
            pl.BlockSpec((BE, 16), lambda i: (i, 0)),
            pl.BlockSpec((NC, BE, H), lambda i: (0, i, 0)),
            pl.BlockSpec((16, G), lambda i: (0, 0)),
            pl.BlockSpec((1, G), lambda i: (0, 0)),
            pl.BlockSpec((P, PW), lambda i: (0, 0)),
            pl.BlockSpec((1, PW), lambda i: (0, 0)),
        ],
        out_specs=[
            pl.BlockSpec((NC, BE, H), lambda i: (0, i, 0)),
            pl.BlockSpec((1, 1, BE), lambda i: (i, 0, 0)),
        ],
        out_shape=[
            jax.ShapeDtypeStruct((NC, E, H), F32),
            jax.ShapeDtypeStruct((NEB, 1, BE), F32),
        ],
        compiler_params=pltpu.CompilerParams(
            dimension_semantics=("arbitrary",)),
    )(ef, npg, wefT, b1, waugT, baug)


def _tcd_body(parts_ref, h_ref, wihT, whhT, bih, bhh, wpnT, bpn, wab, b2,
              ho_ref, hpc_ref, uv_ref):
    pr = parts_ref[...]
    nums = jnp.concatenate([pr[0][:, :PW - H], pr[1]], axis=1)
    ssum = nums[:, 223:224]
    ctx = _elu(nums[:, :G] / jnp.maximum(ssum, 1e-12))
    h = h_ref[...][:, :G]
    gi = ctx @ wihT[...] + bih[...]
    gh = h @ whhT[...] + bhh[...]
    r = jax.nn.sigmoid(gi[:, :G] + gh[:, :G])
    z = jax.nn.sigmoid(gi[:, G:2 * G] + gh[:, G:2 * G])
    nn_ = jnp.tanh(gi[:, 2 * G:] + r * gh[:, 2 * G:])
    hn = jax.nn.relu((1.0 - z) * nn_ + z * h)
    ho_ref[...] = jnp.pad(hn, ((0, 0), (0, 56)))
    hp = hn @ wpnT[...] + bpn[...]
    hpa = jnp.concatenate(
        [hp, jnp.zeros((hn.shape[0], 2 * HA - G - 1), F32),
         jnp.ones((hn.shape[0], 1), F32),
         jnp.zeros((hn.shape[0], PW - 2 * HA), F32)],
        axis=1)
    hpc_ref[...] = jnp.stack([hpa[:, :H], hpa[:, PW - H:]], axis=0)
    q = (lax.dot_general(wab[...], hn, (((0,), (1,)), ((), ())),
                         preferred_element_type=F32) + b2[...])
    uv_ref[...] = jnp.concatenate([q[0:1], q[1:2]], axis=1)[None]


def _tcd(parts, h, wihT, whhT, bih, bhh, wpnT, bpn, wab, b2):
    return pl.pallas_call(
        _tcd_body,
        grid=(NB,),
        in_specs=[
            pl.BlockSpec((NC, BN, H), lambda i: (0, i, 0)),
            pl.BlockSpec((BN, 256), lambda i: (i, 0)),
            pl.BlockSpec((G, 3 * G), lambda i: (0, 0)),
            pl.BlockSpec((G, 3 * G), lambda i: (0, 0)),
            pl.BlockSpec((1, 3 * G), lambda i: (0, 0)),
            pl.BlockSpec((1, 3 * G), lambda i: (0, 0)),
            pl.BlockSpec((G, G), lambda i: (0, 0)),
            pl.BlockSpec((1, G), lambda i: (0, 0)),
            pl.BlockSpec((G, 2), lambda i: (0, 0)),
            pl.BlockSpec((2, 1), lambda i: (0, 0)),
        ],
        out_specs=[
            pl.BlockSpec((BN, 256), lambda i: (i, 0)),
            pl.BlockSpec((NC, BN, H), lambda i: (0, i, 0)),
            pl.BlockSpec((1, 1, 2 * BN), lambda i: (i, 0, 0)),
        ],
        out_shape=[
            jax.ShapeDtypeStruct((N, 256), F32),
            jax.ShapeDtypeStruct((NC, N, H), F32),
            jax.ShapeDtypeStruct((NB, 1, 2 * BN), F32),
        ],
        compiler_params=pltpu.CompilerParams(
            dimension_semantics=("arbitrary",)),
    )(parts, h, wihT, whhT, bih, bhh, wpnT, bpn, wab, b2)


def _tcr_body(ids_ref, h_ref, wa2, wb2, bt2, wpnT2, bpn2, wihT2, whhT2,
              bih2, bhh2, w1T, b1, w2Tp, b2p, out_ref, g_sc, an_sc, as_sc):
    p = pl.program_id(0)
    b = pl.program_id(1)
    h = h_ref[...][:, :G]
    ids = ids_ref[0, 0]
    M = (ids[:, None] == lax.broadcasted_iota(jnp.int32, (BN, NG), 1)
         ).astype(F32)

    @pl.when(jnp.logical_and(p == 0, b == 0))
    def _():
        g_sc[...] = jnp.zeros((NG, 256), F32)

    @pl.when(p == 0)
    def _():
        gb = lax.dot_general(M, h, (((0,), (0,)), ((), ())),
                             preferred_element_type=F32)
        g_sc[...] += jnp.pad(gb, ((0, 0), (0, 56)))

    @pl.when(p > 0)
    def _():
        t_is0 = p == 1

        def pick(w2):
            w = w2[...]
            return jnp.where(t_is0, w[0], w[1])

        wa = pick(wa2)
        wb = pick(wb2)
        bt = pick(bt2)
        wpnT = pick(wpnT2)
        bpn = pick(bpn2)

        @pl.when(b == 0)
        def _():
            an_sc[...] = jnp.zeros((NG, 256), F32)
            as_sc[...] = jnp.zeros((NG, 8), F32)

        g = g_sc[...][:, :G]
        ra = jax.nn.relu(g) @ wa
        raN = M @ ra[:, 0:1]
        zlog = _leaky(raN + (h @ wb)[:, 0:1] + bt[0, 0])
        ez = jnp.exp(zlog)
        hvp = h @ wpnT + bpn
        an_sc[...] += jnp.pad(
            lax.dot_general(M, ez * hvp, (((0,), (0,)), ((), ())),
                            preferred_element_type=F32),
            ((0, 0), (0, 56)))
        as_sc[:, 0:1] += lax.dot_general(M, ez, (((0,), (0,)), ((), ())),
                                         preferred_element_type=F32)

        @pl.when(b == NB - 1)
        def _():
            wihT = pick(wihT2)
            whhT = pick(whhT2)
            bih = pick(bih2)
            bhh = pick(bhh2)
            s = jnp.maximum(as_sc[...][:, 0:1], 1e-12)
            grp = an_sc[...][:, :G] / s
            gin = _elu(grp)
            gprev = g_sc[...][:, :G]
            gi = gin @ wihT + bih
            gh = gprev @ whhT + bhh
            r = jax.nn.sigmoid(gi[:, :G] + gh[:, :G])
            zz = jax.nn.sigmoid(gi[:, G:2 * G] + gh[:, G:2 * G])
            nn_ = jnp.tanh(gi[:, 2 * G:] + r * gh[:, 2 * G:])
            gnew = jax.nn.relu((1.0 - zz) * nn_ + zz * gprev)
            g_sc[...] = jnp.pad(gnew, ((0, 0), (0, 56)))

            @pl.when(p == 2)
            def _():
                hidden = jax.nn.relu(gnew @ w1T[...] + b1[...])
                out_ref[...] = hidden @ w2Tp[...] + b2p[...]


def _tcr(ids3, h, args):
    def full(s):
        return pl.BlockSpec(s, lambda p, b: tuple(0 for _ in s))

    return pl.pallas_call(
        _tcr_body,
        grid=(3, NB),
        in_specs=[
            pl.BlockSpec((1, 1, BN), lambda p, b: (b, 0, 0)),
            pl.BlockSpec((BN, 256), lambda p, b: (b, 0)),
            full((2, G, 8)), full((2, G, 8)), full((2, 1, 8)),
            full((2, G, G)), full((2, 1, G)),
            full((2, G, 3 * G)), full((2, G, 3 * G)),
            full((2, 1, 3 * G)), full((2, 1, 3 * G)),
            full((G, 1024)), full((1, 1024)),
            full((1024, 128)), full((1, 128)),
        ],
        out_specs=pl.BlockSpec((NG, 128), lambda p, b: (0, 0)),
        out_shape=jax.ShapeDtypeStruct((NG, 128), F32),
        scratch_shapes=[
            pltpu.VMEM((NG, 256), F32),
            pltpu.VMEM((NG, 256), F32),
            pltpu.VMEM((NG, 8), F32),
        ],
        compiler_params=pltpu.CompilerParams(
            dimension_semantics=("arbitrary", "arbitrary")),
    )(ids3, h, *args)


_MESH = plsc.VectorSubcoreMesh(core_axis_name="c", subcore_axis_name="s")


@functools.partial(
    pl.kernel,
    out_type=jax.ShapeDtypeStruct((NC, E, H), F32),
    mesh=_MESH,
    compiler_params=pltpu.CompilerParams(needs_layout_passes=False),
    scratch_types=[
        pltpu.VMEM((NCK, CH), jnp.int32),
        pltpu.VMEM((CH, H), F32),
        pltpu.VMEM((CH, H), F32),
        pltpu.SemaphoreType.DMA,
        pltpu.SemaphoreType.DMA,
    ],
)
def _sc_gather(srcp, tab, out, idx_v, bufA, bufB, semA, semB):
    c = lax.axis_index("c")
    s = lax.axis_index("s")
    pltpu.sync_copy(srcp.at[s], idx_v)
    cN = (c * N).astype(jnp.int32)

    def adj(j, _):
        for k in range(CH // 16):
            sl = (j, pl.ds(k * 16, 16))
            idx_v[sl] = idx_v[sl] + cN
        return 0

    lax.fori_loop(0, NCK, adj, 0, unroll=False)
    ebase = s * EPT

    def issue(j, buf, sem):
        pltpu.async_copy(tab.at[idx_v.at[j]], buf, sem)

    def wait(buf, sem):
        pltpu.make_async_copy(tab.at[pl.ds(0, CH)], buf, sem).wait()

    def flush(j, buf):
        pltpu.sync_copy(buf, out.at[c, pl.ds(ebase + j * CH, CH)])

    issue(0, bufA, semA)
    issue(1, bufB, semB)

    def body(jj, _):
        j0 = 2 * jj
        wait(bufA, semA)
        flush(j0, bufA)

        @pl.when(j0 + 2 < NCK)
        def _():
            issue(j0 + 2, bufA, semA)

        wait(bufB, semB)
        flush(j0 + 1, bufB)

        @pl.when(j0 + 3 < NCK)
        def _():
            issue(j0 + 3, bufB, semB)

        return 0

    lax.fori_loop(0, NCK // 2, body, 0, unroll=False)


_SPMM_SCRATCH = [
    pltpu.VMEM((4, CH), jnp.int32),
    pltpu.VMEM((2, CH), F32),
    pltpu.VMEM((2, CH), jnp.int32),
    pltpu.VMEM((N + BN + 16,), F32),
    pltpu.VMEM((N + BN + 16,), F32),
    pltpu.VMEM((16,), jnp.int32),
    pltpu.VMEM((CH,), F32),
    pltpu.VMEM((CH, H), F32),
    pltpu.VMEM((CH, H), F32),
    pltpu.VMEM_SHARED((N, H), F32),
    pltpu.SemaphoreType.DMA,
    pltpu.SemaphoreType.DMA,
    pltpu.SemaphoreType.DMA,
    pltpu.SemaphoreType.DMA,
]


@functools.partial(
    pl.kernel,
    out_type=jax.ShapeDtypeStruct((NC, N, H), F32),
    mesh=_MESH,
    compiler_params=pltpu.CompilerParams(needs_layout_passes=False),
    scratch_types=_SPMM_SCRATCH,
)
def _spmm(ridx, wlin, flg, dstp, uvp, tab, out,
          idxb, wl2, dst2, u_v, v_v, fl_v, e_v, bufA, bufB,
          acc, semA, semB, semI, semX):
    c = lax.axis_index("c")
    s = lax.axis_index("s")
    lane = lax.iota(jnp.int32, 16)
    zf16 = (lane * 0).astype(F32)
    pltpu.sync_copy(flg, fl_v)
    for b in range(NB):
        pltpu.sync_copy(uvp.at[b, 0], u_v.at[pl.ds(b * BN, 2 * BN)])
    for b in reversed(range(NB)):
        pltpu.sync_copy(uvp.at[b, 0], v_v.at[pl.ds(b * BN, 2 * BN)])
    cE = (c * E).astype(jnp.int32)

    def zr(i, _):
        for m in range(H // 16):
            bufA[i, pl.ds(m * 16, 16)] = zf16
        return 0

    lax.fori_loop(0, CH, zr, 0, unroll=False)

    @pl.when(s < NB)
    def _():
        for r in range(12):
            pltpu.sync_copy(bufA, acc.at[pl.ds(s * BN + r * CH, CH)])
        pltpu.sync_copy(bufA.at[pl.ds(0, 40)],
                        acc.at[pl.ds(s * BN + 12 * CH, 40)])
    plsc.subcore_barrier()

    def issue_idx(j):
        pltpu.async_copy(ridx.at[s, j], idxb.at[j % 4], semX)

    def issue_row(j, buf, sem):
        pltpu.make_async_copy(ridx.at[s, 0], idxb.at[j % 4], semX).wait()
        for k in range(CH // 16):
            sl = (j % 4, pl.ds(k * 16, 16))
            idxb[sl] = idxb[sl] + cE
        pltpu.async_copy(tab.at[idxb.at[j % 4]], buf, sem)
        pltpu.async_copy(dstp.at[s, j], dst2.at[j % 2], semI)
        pltpu.async_copy(wlin.at[s, j], wl2.at[j % 2], semI)

    def wait_row(j, buf, sem):
        pltpu.make_async_copy(tab.at[pl.ds(0, CH)], buf, sem).wait()
        pltpu.make_async_copy(dstp.at[s, 0], dst2.at[j % 2], semI).wait()
        pltpu.make_async_copy(wlin.at[s, 0], wl2.at[j % 2], semI).wait()

    def process(j, buf):
        pb = j % 2
        fv = fl_v[...] > 0
        for k in range(CH // 16):
            slk = pl.ds(k * 16, 16)
            idd = dst2[pb, slk]
            uu = plsc.load_gather(u_v, [idd])
            gidx = jnp.minimum(idxb[j % 4, slk] - cE, N - 1) + BN
            vg = plsc.load_gather(v_v, [gidx])
            vv = jnp.where(fv, wl2[pb, slk], vg)
            lg = uu + vv
            e_v[slk] = jnp.exp(jnp.maximum(lg, 0.01 * lg))

        def srow(i, _):
            ei = plsc.load_gather(
                e_v, [jnp.broadcast_to(i, (16,)).astype(jnp.int32)])
            for m in range(H // 16):
                sl = (i, pl.ds(m * 16, 16))
                buf[sl] = buf[sl] * ei
            return 0

        lax.fori_loop(0, CH, srow, 0, unroll=False)
        pltpu.sync_copy(buf, acc.at[dst2.at[pb]], add=True)

    issue_idx(0)
    issue_idx(1)
    issue_row(0, bufA, semA)
    issue_idx(2)
    issue_row(1, bufB, semB)
    issue_idx(3)

    def body(jj, _):
        j0 = 2 * jj
        wait_row(j0, bufA, semA)
        process(j0, bufA)

        @pl.when(j0 + 2 < NCK)
        def _():
            issue_row(j0 + 2, bufA, semA)

        @pl.when(j0 + 4 < NCK)
        def _():
            issue_idx(j0 + 4)

        wait_row(j0 + 1, bufB, semB)
        process(j0 + 1, bufB)

        @pl.when(j0 + 3 < NCK)
        def _():
            issue_row(j0 + 3, bufB, semB)

        @pl.when(j0 + 5 < NCK)
        def _():
            issue_idx(j0 + 5)

        return 0

    lax.fori_loop(0, NCK // 2, body, 0, unroll=False)
    plsc.subcore_barrier()

    @pl.when(s < NB)
    def _():
        for r in range(12):
            pltpu.sync_copy(acc.at[pl.ds(s * BN + r * CH, CH)], bufA)
            pltpu.sync_copy(bufA, out.at[c, pl.ds(s * BN + r * CH, CH)])
        pltpu.sync_copy(acc.at[pl.ds(s * BN + 12 * CH, 40)],
                        bufA.at[pl.ds(0, 40)])
        pltpu.sync_copy(bufA.at[pl.ds(0, 40)],
                        out.at[c, pl.ds(s * BN + 12 * CH, 40)])


def _row(x):
    return x.reshape(1, -1)


def kernel(node_feat, edge_feat, edge_index, node_graph_ids, params):
    p = params
    src = edge_index[0]
    dst = edge_index[1]
    srcp = src.reshape(NS, NCK, CH)
    dstp = dst.reshape(NS, NCK, CH)
    eidp = jnp.arange(E, dtype=jnp.int32).reshape(NS, NCK, CH)

    wa_ctx = p['ctx_W_pe2'][0, :G]
    wb_ctx = p['ctx_W_pe2'][0, G:]
    b_ctx2 = p['ctx_b_pe2'][0]
    wab_ctx = jnp.stack([wa_ctx, jnp.zeros((G,), F32)], axis=1)
    waugT = jnp.zeros((P, PW), F32)
    waugT = waugT.at[:G, :G].set(p['ctx_W_et'].T)
    waugT = waugT.at[:G, 222].set(wb_ctx)
    baug = jnp.zeros((PW,), F32)
    baug = baug.at[:G].set(p['ctx_b_et'])
    baug = baug.at[222].set(b_ctx2)
    baug = baug.at[223].set(1.0)

    hv, npc, uc = _tca(node_feat,
                       p['ctx_W_pn'].T, _row(p['ctx_b_pn']),
                       p['ctx_W_pe1'][:, :128].T, wab_ctx)
    npg = _sc_gather(srcp, npc.reshape(NC * N, H))
    et, sbp = _tcc(edge_feat, npg,
                   p['ctx_W_pe1'][:, 128:].T, _row(p['ctx_b_pe1']),
                   waugT, _row(baug))
    sbp = sbp.reshape(NS, NCK, CH)

    ridx_s = jnp.concatenate([eidp[None]] + [srcp[None]] * 4, axis=0)
    wlin_s = jnp.concatenate([sbp[None], jnp.zeros((4, NS, NCK, CH), F32)], axis=0)
    flg_s = jnp.concatenate([jnp.ones((1, 16), jnp.int32),
                             jnp.zeros((4, 16), jnp.int32)], axis=0)

    def stk(key_c, key_g, tr):
        c = p[key_c].T if tr else p[key_c]
        gs = [p[key_g][l].T if tr else p[key_g][l] for l in range(4)]
        return jnp.stack([c] + gs, axis=0)

    wihT_s = stk('ctx_Wih', 'gnn_Wih', True)
    whhT_s = stk('ctx_Whh', 'gnn_Whh', True)
    bih_s = jnp.stack([_row(p['ctx_bih'])] + [_row(p['gnn_bih'][l]) for l in range(4)])
    bhh_s = jnp.stack([_row(p['ctx_bhh'])] + [_row(p['gnn_bhh'][l]) for l in range(4)])

    def prep(l):
        wa = p['gnn_W_pe'][l][0, :G]
        wb = p['gnn_W_pe'][l][0, G:]
        b = p['gnn_b_pe'][l][0]
        wab = jnp.stack([wa, wb], axis=1)
        b2 = jnp.stack([jnp.zeros((), F32), b]).reshape(2, 1)
        return p['gnn_W_pn'][l].T, _row(p['gnn_b_pn'][l]), wab, b2

    preps = [prep(l) for l in [0, 1, 2, 3, 0]]
    wpnT_s = jnp.stack([q[0] for q in preps])
    bpn_s = jnp.stack([q[1] for q in preps])
    wab_s = jnp.stack([q[2] for q in preps])
    b2_s = jnp.stack([q[3] for q in preps])

    tab0 = et.reshape(NC * E, H)

    def step(carry, xs):
        h, uvp, tab = carry
        ridx, wlin, flg, wihT, whhT, bih, bhh, wpnT, bpn, wab, b2 = xs
        parts = _spmm(ridx, wlin, flg, dstp, uvp, tab)
        h2, hpc, uv = _tcd(parts, h, wihT, whhT, bih, bhh, wpnT, bpn, wab, b2)
        tab = lax.dynamic_update_slice(tab, hpc[0], (0, 0))
        tab = lax.dynamic_update_slice(tab, hpc[1], (E, 0))
        return (h2, uv, tab), 0.0

    (h, _, _), _ = lax.scan(
        step, (hv, uc, tab0),
        (ridx_s, wlin_s, flg_s, wihT_s, whhT_s, bih_s, bhh_s,
         wpnT_s, bpn_s, wab_s, b2_s))

    def pad8(v):
        return jnp.pad(v.reshape(-1, 1), ((0, 0), (0, 7)))

    ids3 = node_graph_ids.reshape(NB, 1, BN)
    ro_args = (
        jnp.stack([pad8(p['ro_W_cl'][t][0, :G]) for t in range(2)]),
        jnp.stack([pad8(p['ro_W_cl'][t][0, G:]) for t in range(2)]),
        jnp.stack([pad8(p['ro_b_cl'][t]) for t in range(2)]),
        jnp.stack([p['ro_W_pn'][t].T for t in range(2)]),
        jnp.stack([_row(p['ro_b_pn'][t]) for t in range(2)]),
        jnp.stack([p['ro_Wih'][t].T for t in range(2)]),
        jnp.stack([p['ro_Whh'][t].T for t in range(2)]),
        jnp.stack([_row(p['ro_bih'][t]) for t in range(2)]),
        jnp.stack([_row(p['ro_bhh'][t]) for t in range(2)]),
        p['out_W1'].T, _row(p['out_b1']),
        jnp.pad(p['out_W2'].T, ((0, 0), (0, 127))),
        jnp.pad(_row(p['out_b2']), ((0, 0), (0, 127))),
    )
    out128 = _tcr(ids3, h, ro_args)
    return out128[:, :1]
